# baseline (speedup 1.0000x reference)
import functools

import jax
import jax.numpy as jnp
from jax import lax
from jax.experimental import pallas as pl
from jax.experimental.pallas import tpu as pltpu
from jax.experimental.pallas import tpu_sc as plsc

B = 16384
MF_DIM = 8
D_UMLP = 16
D_IMLP = 16
D_KMLP = 32
L1 = 32
L2 = 16


def _sc_gather5(u_idx, i_idx, k_idx, e_umf, e_imf, e_umlp, e_imlp, e_kmlp):
    info = plsc.get_sparse_core_info()
    nw = info.num_cores * info.num_subcores
    bw = B // nw

    mesh = plsc.VectorSubcoreMesh(core_axis_name="c", subcore_axis_name="s")
    out_type = (
        jax.ShapeDtypeStruct((B, MF_DIM), jnp.float32),
        jax.ShapeDtypeStruct((B, MF_DIM), jnp.float32),
        jax.ShapeDtypeStruct((B, D_UMLP), jnp.float32),
        jax.ShapeDtypeStruct((B, D_IMLP), jnp.float32),
        jax.ShapeDtypeStruct((B, D_KMLP), jnp.float32),
    )
    scratch = [
        pltpu.VMEM((bw,), jnp.int32),
        pltpu.VMEM((bw,), jnp.int32),
        pltpu.VMEM((bw,), jnp.int32),
        pltpu.VMEM((bw, MF_DIM), jnp.float32),
        pltpu.VMEM((bw, MF_DIM), jnp.float32),
        pltpu.VMEM((bw, D_UMLP), jnp.float32),
        pltpu.VMEM((bw, D_IMLP), jnp.float32),
        pltpu.VMEM((bw, D_KMLP), jnp.float32),
        pltpu.SemaphoreType.DMA,
        pltpu.SemaphoreType.DMA,
        pltpu.SemaphoreType.DMA,
        pltpu.SemaphoreType.DMA,
        pltpu.SemaphoreType.DMA,
    ]

    @functools.partial(
        pl.kernel, out_type=out_type, mesh=mesh, scratch_types=scratch,
        compiler_params=pltpu.CompilerParams(use_tc_tiling_on_sc=False),
    )
    def gather_kernel(u_hbm, i_hbm, k_hbm, t_umf, t_imf, t_umlp, t_imlp,
                      t_kmlp, o_umf, o_imf, o_umlp, o_imlp, o_kmlp,
                      iu, ii, ik, ru, ri, rum, rim, rkm, s0, s1, s2, s3, s4):
        wid = lax.axis_index("s") * info.num_cores + lax.axis_index("c")
        base = wid * bw
        pltpu.sync_copy(u_hbm.at[pl.ds(base, bw)], iu)
        pltpu.sync_copy(i_hbm.at[pl.ds(base, bw)], ii)
        pltpu.sync_copy(k_hbm.at[pl.ds(base, bw)], ik)
        c0 = pltpu.async_copy(t_umf.at[iu], ru, s0)
        c1 = pltpu.async_copy(t_imf.at[ii], ri, s1)
        c2 = pltpu.async_copy(t_umlp.at[iu], rum, s2)
        c3 = pltpu.async_copy(t_imlp.at[ii], rim, s3)
        c4 = pltpu.async_copy(t_kmlp.at[ik], rkm, s4)
        c0.wait()
        pltpu.sync_copy(ru, o_umf.at[pl.ds(base, bw)])
        c1.wait()
        pltpu.sync_copy(ri, o_imf.at[pl.ds(base, bw)])
        c2.wait()
        pltpu.sync_copy(rum, o_umlp.at[pl.ds(base, bw)])
        c3.wait()
        pltpu.sync_copy(rim, o_imlp.at[pl.ds(base, bw)])
        c4.wait()
        pltpu.sync_copy(rkm, o_kmlp.at[pl.ds(base, bw)])

    return gather_kernel(u_idx, i_idx, k_idx, e_umf, e_imf, e_umlp, e_imlp,
                         e_kmlp)


def _mlp_body(umf, imf, umlp, imlp, kmlp, w1, b1, w2, b2, wp, bp, out):
    f32 = jnp.float32
    w1v = w1[...]
    h1 = (
        jnp.dot(umlp[...], w1v[0:D_UMLP, :], preferred_element_type=f32)
        + jnp.dot(imlp[...], w1v[D_UMLP:D_UMLP + D_IMLP, :],
                  preferred_element_type=f32)
        + jnp.dot(kmlp[...], w1v[D_UMLP + D_IMLP:, :],
                  preferred_element_type=f32)
        + b1[...]
    )
    h1 = jnp.maximum(h1, 0.0)
    h2 = jnp.maximum(
        jnp.dot(h1, w2[...], preferred_element_type=f32) + b2[...], 0.0)
    mf = umf[...] * imf[...]
    wpv = wp[...]
    logits = (
        jnp.dot(mf, wpv[0:MF_DIM, :], preferred_element_type=f32)
        + jnp.dot(h2, wpv[MF_DIM:, :], preferred_element_type=f32)
        + bp[...]
    )
    out[...] = jax.nn.sigmoid(logits[:, 0])


def _tc_mlp(umf, imf, umlp, imlp, kmlp, w1, b1, w2, b2, wp, bp):
    nb = 2048
    grid = (B // nb,)

    def row_spec(d):
        return pl.BlockSpec((nb, d), lambda i: (i, 0))

    def full_spec(shape):
        return pl.BlockSpec(shape, lambda i: tuple(0 for _ in shape))

    return pl.pallas_call(
        _mlp_body,
        grid=grid,
        in_specs=[
            row_spec(MF_DIM),
            row_spec(MF_DIM),
            row_spec(D_UMLP),
            row_spec(D_IMLP),
            row_spec(D_KMLP),
            full_spec((D_UMLP + D_IMLP + D_KMLP, L1)),
            full_spec((1, L1)),
            full_spec((L1, L2)),
            full_spec((1, L2)),
            full_spec((MF_DIM + L2, 1)),
            full_spec((1, 1)),
        ],
        out_specs=pl.BlockSpec((nb,), lambda i: (i,)),
        out_shape=jax.ShapeDtypeStruct((B,), jnp.float32),
    )(umf, imf, umlp, imlp, kmlp, w1, b1, w2, b2, wp, bp)


def kernel(user_indices, item_indices, kg_indices, E_user_mf, E_item_mf,
           E_user_mlp, E_item_mlp, E_kg_mlp, W1, b1, W2, b2, Wp, bp):
    umf, imf, umlp, imlp, kmlp = _sc_gather5(
        user_indices, item_indices, kg_indices,
        E_user_mf, E_item_mf, E_user_mlp, E_item_mlp, E_kg_mlp)
    return _tc_mlp(umf, imf, umlp, imlp, kmlp, W1,
                   b1.reshape(1, L1), W2, b2.reshape(1, L2), Wp,
                   bp.reshape(1, 1))
